# baseline (speedup 1.0000x reference)
import functools

import jax
import jax.numpy as jnp
from jax import lax
from jax.experimental import pallas as pl
from jax.experimental.pallas import tpu as pltpu
from jax.experimental.pallas import tpu_sc as plsc

B = 16384
F = 26
V = 100000
D = 32
FLOAT_DIM = 128
NLP_DIM = 768
OUTC = F + 3

NC = 2
NS = 16
NW = NC * NS
BPW = B // NW
RW = BPW * F
IL = 128
GPW = RW // IL
MC = 8
NM = GPW // MC
PGW = BPW * 3 // IL

_mesh = plsc.VectorSubcoreMesh(
    core_axis_name="c", subcore_axis_name="s", num_cores=NC, num_subcores=NS
)


@functools.partial(
    pl.kernel,
    out_type=jax.ShapeDtypeStruct((B * OUTC, D), jnp.float32),
    mesh=_mesh,
    scratch_types=[
        pltpu.VMEM((GPW, IL), jnp.int32),
        pltpu.VMEM((GPW, IL), jnp.int32),
        pltpu.VMEM((IL,), jnp.int32),
        pltpu.VMEM((IL, D), jnp.float32),
        pltpu.VMEM((MC, IL, D), jnp.float32),
        pltpu.SemaphoreType.DMA,
        pltpu.SemaphoreType.DMA,
    ],
    compiler_params=pltpu.CompilerParams(use_tc_tiling_on_sc=False),
)
def _sc_gather(table_ref, idx_ref, proj_ref, out_ref,
               idxv, obase, pidx, projv, rows, gsem, ssem):
    wid = lax.axis_index("s") * NC + lax.axis_index("c")
    b0 = wid * BPW
    pltpu.sync_copy(idx_ref.at[pl.ds(wid * GPW, GPW)], idxv)

    def idx_body(j, carry):
        for l in range(IL // 16):
            p = j * IL + l * 16
            ids = lax.iota(jnp.int32, 16) + p
            bvec = lax.div(ids, jnp.int32(F))
            fvec = ids - bvec * F
            sl = pl.ds(l * 16, 16)
            idxv[j, sl] = idxv[j, sl] + fvec * V
            obase[j, sl] = (bvec + b0) * OUTC + fvec
        return carry
    lax.fori_loop(0, GPW, idx_body, 0)

    def macro_body(m, carry):
        g0 = m * MC
        hs = [
            pltpu.async_copy(table_ref.at[idxv.at[g0 + g]], rows.at[g], gsem)
            for g in range(MC)
        ]
        for h in hs:
            h.wait()
        hs = [
            pltpu.async_copy(rows.at[g], out_ref.at[obase.at[g0 + g]], ssem)
            for g in range(MC)
        ]
        for h in hs:
            h.wait()
        return carry
    lax.fori_loop(0, NM, macro_body, 0)

    def proj_body(m, carry):
        pltpu.sync_copy(proj_ref.at[pl.ds(b0 * 3 + m * IL, IL)], projv)
        for l in range(IL // 16):
            p = m * IL + l * 16
            ids = lax.iota(jnp.int32, 16) + p
            bvec = lax.div(ids, jnp.int32(3))
            cvec = ids - bvec * 3
            pidx[pl.ds(l * 16, 16)] = (b0 + bvec) * OUTC + F + cvec
        pltpu.sync_copy(projv, out_ref.at[pidx])
        return carry
    lax.fori_loop(0, PGW, proj_body, 0)


_BB = 2048


def _proj_kernel(f_ref, c_ref, s_ref, wf, bf, wc, bc, ws, bs, o_ref):
    o_ref[:, 0:D] = (
        jnp.dot(f_ref[...], wf[...], preferred_element_type=jnp.float32) + bf[...]
    )
    o_ref[:, D:2 * D] = (
        jnp.dot(c_ref[...], wc[...], preferred_element_type=jnp.float32) + bc[...]
    )
    o_ref[:, 2 * D:3 * D] = (
        jnp.dot(s_ref[...], ws[...], preferred_element_type=jnp.float32) + bs[...]
    )


_tc_proj = pl.pallas_call(
    _proj_kernel,
    grid=(B // _BB,),
    in_specs=[
        pl.BlockSpec((_BB, FLOAT_DIM), lambda i: (i, 0)),
        pl.BlockSpec((_BB, NLP_DIM), lambda i: (i, 0)),
        pl.BlockSpec((_BB, NLP_DIM), lambda i: (i, 0)),
        pl.BlockSpec((FLOAT_DIM, D), lambda i: (0, 0)),
        pl.BlockSpec((1, D), lambda i: (0, 0)),
        pl.BlockSpec((NLP_DIM, D), lambda i: (0, 0)),
        pl.BlockSpec((1, D), lambda i: (0, 0)),
        pl.BlockSpec((NLP_DIM, D), lambda i: (0, 0)),
        pl.BlockSpec((1, D), lambda i: (0, 0)),
    ],
    out_specs=pl.BlockSpec((_BB, 3 * D), lambda i: (i, 0)),
    out_shape=jax.ShapeDtypeStruct((B, 3 * D), jnp.float32),
)


@jax.jit
def kernel(float_inputs, idx_inputs, comment_vecs, spotlight_vecs, tables,
           W_float, b_float, W_comm, b_comm, W_spot, b_spot):
    proj = _tc_proj(
        float_inputs, comment_vecs, spotlight_vecs,
        W_float, b_float.reshape(1, D),
        W_comm, b_comm.reshape(1, D),
        W_spot, b_spot.reshape(1, D),
    )
    proj_flat = proj.reshape(B * 3, D)
    table_flat = tables.reshape(F * V, D)
    idx2 = idx_inputs.astype(jnp.int32).reshape(B * F // IL, IL)
    out_flat = _sc_gather(table_flat, idx2, proj_flat)
    return out_flat.reshape(B, OUTC * D)
